# Initial kernel scaffold; baseline (speedup 1.0000x reference)
import jax
import jax.numpy as jnp
from jax.experimental import pallas as pl


def kernel(x, prototypes):
    raise NotImplementedError("write your pallas kernel here")


# speedup vs baseline: 1.0119x; 1.0119x over previous
import functools

import jax
import jax.numpy as jnp
from jax import lax
from jax.experimental import pallas as pl
from jax.experimental.pallas import tpu as pltpu

_TOPK = 5
_ALPHA = 0.1
_EPS2 = 1e-24


def _sum_body(x_ref, o_ref):
    @pl.when(pl.program_id(0) == 0)
    def _init():
        o_ref[...] = jnp.zeros_like(o_ref)

    o_ref[...] += jnp.sum(x_ref[...], axis=1)


def _scores_body(q_ref, p_ref, s_ref, *, seq_len):
    q = q_ref[...] / seq_len
    qn = q * lax.rsqrt(jnp.maximum(jnp.sum(q * q, axis=1, keepdims=True), _EPS2))
    p = p_ref[...]
    pn2 = jnp.sum(p * p, axis=1)
    dots = lax.dot_general(
        qn, p, (((1,), (1,)), ((), ())), preferred_element_type=jnp.float32
    )
    s_ref[...] = dots * lax.rsqrt(jnp.maximum(pn2, _EPS2))[None, :]


def _topk_weights(s):
    bsz, k = s.shape
    colid = lax.broadcasted_iota(jnp.int32, (bsz, k), 1)
    work = s
    vals, idxs = [], []
    for _ in range(_TOPK):
        m = jnp.max(work, axis=1, keepdims=True)
        idx = jnp.min(jnp.where(work == m, colid, k), axis=1, keepdims=True)
        vals.append(m)
        idxs.append(idx)
        work = jnp.where(colid == idx, -jnp.inf, work)
    v = jnp.concatenate(vals, axis=1)
    e = jnp.exp(v - v[:, :1])
    w = e / jnp.sum(e, axis=1, keepdims=True)
    return w, idxs


def _agg_body(s_ref, p_ref, o_ref, *, kb):
    i = pl.program_id(0)

    @pl.when(i == 0)
    def _init():
        o_ref[...] = jnp.zeros_like(o_ref)

    w, idxs = _topk_weights(s_ref[...])
    bsz = w.shape[0]
    colid = i * kb + lax.broadcasted_iota(jnp.int32, (bsz, kb), 1)
    wblk = jnp.zeros((bsz, kb), jnp.float32)
    for j in range(_TOPK):
        wblk = wblk + jnp.where(colid == idxs[j], w[:, j : j + 1], 0.0)
    o_ref[...] += jnp.dot(wblk, p_ref[...], preferred_element_type=jnp.float32)


def _add_body(x_ref, a_ref, o_ref):
    o_ref[...] = x_ref[...] + _ALPHA * a_ref[...][:, None, :]


def _pipeline(x, prototypes, interpret=False):
    bsz, seq_len, d = x.shape
    k = prototypes.shape[0]
    lb = 512
    kb = 1024

    qsum = pl.pallas_call(
        _sum_body,
        grid=(seq_len // lb,),
        in_specs=[pl.BlockSpec((bsz, lb, d), lambda i: (0, i, 0))],
        out_specs=pl.BlockSpec((bsz, d), lambda i: (0, 0)),
        out_shape=jax.ShapeDtypeStruct((bsz, d), jnp.float32),
        interpret=interpret,
    )(x)

    scores = pl.pallas_call(
        functools.partial(_scores_body, seq_len=seq_len),
        grid=(k // kb,),
        in_specs=[
            pl.BlockSpec((bsz, d), lambda i: (0, 0)),
            pl.BlockSpec((kb, d), lambda i: (i, 0)),
        ],
        out_specs=pl.BlockSpec((bsz, kb), lambda i: (0, i)),
        out_shape=jax.ShapeDtypeStruct((bsz, k), jnp.float32),
        interpret=interpret,
    )(qsum, prototypes)

    agg = pl.pallas_call(
        functools.partial(_agg_body, kb=kb),
        grid=(k // kb,),
        in_specs=[
            pl.BlockSpec((bsz, k), lambda i: (0, 0)),
            pl.BlockSpec((kb, d), lambda i: (i, 0)),
        ],
        out_specs=pl.BlockSpec((bsz, d), lambda i: (0, 0)),
        out_shape=jax.ShapeDtypeStruct((bsz, d), jnp.float32),
        interpret=interpret,
    )(scores, prototypes)

    out = pl.pallas_call(
        _add_body,
        grid=(seq_len // lb,),
        in_specs=[
            pl.BlockSpec((bsz, lb, d), lambda i: (0, i, 0)),
            pl.BlockSpec((bsz, d), lambda i: (0, 0)),
        ],
        out_specs=pl.BlockSpec((bsz, lb, d), lambda i: (0, i, 0)),
        out_shape=jax.ShapeDtypeStruct((bsz, seq_len, d), jnp.float32),
        interpret=interpret,
    )(x, agg)
    return out


def kernel(x, prototypes):
    return _pipeline(x, prototypes)
